# baseline (speedup 1.0000x reference)
import functools

import jax
import jax.numpy as jnp
from jax import lax
from jax.experimental import pallas as pl
from jax.experimental.pallas import tpu as pltpu
from jax.experimental.pallas import tpu_sc as plsc

_D = 64
_BB = 4096
_S = 50

_info = plsc.get_sparse_core_info()
_NC, _NS = _info.num_cores, _info.num_subcores
_NW = _NC * _NS
_RPW = _BB // _NW
_CR = 8
_NCHUNK = _RPW // _CR
_NBUF = 4
_LOOK = 2


@functools.partial(
    pl.kernel,
    mesh=plsc.VectorSubcoreMesh(core_axis_name="c", subcore_axis_name="s"),
    out_type=jax.ShapeDtypeStruct((_BB, _S, _D), jnp.float32),
    scratch_types=[
        pltpu.VMEM((_RPW, _S), jnp.int32),
        *[pltpu.VMEM((_CR, _S, _D), jnp.float32) for _ in range(_NBUF)],
        *[pltpu.SemaphoreType.DMA for _ in range(2 * _NBUF)],
    ],
    compiler_params=pltpu.CompilerParams(use_tc_tiling_on_sc=False),
)
def _gather_kernel(idx_hbm, table_hbm, out_hbm, idx_v, *scratch):
    bufs = scratch[:_NBUF]
    gsems = scratch[_NBUF:2 * _NBUF]
    osems = scratch[2 * _NBUF:]
    wid = lax.axis_index("s") * _NC + lax.axis_index("c")
    base = wid * _RPW
    pltpu.sync_copy(idx_hbm.at[pl.ds(base, _RPW)], idx_v)
    gathers = {}
    outs = {}
    for t in range(_NCHUNK + _LOOK):
        if t < _NCHUNK:
            b = t % _NBUF
            if t >= _NBUF:
                outs[t - _NBUF].wait()
            gathers[t] = [
                pltpu.async_copy(
                    table_hbm.at[idx_v.at[t * _CR + j]],
                    bufs[b].at[j],
                    gsems[b])
                for j in range(_CR)
            ]
        d = t - _LOOK
        if 0 <= d < _NCHUNK:
            for g in gathers[d]:
                g.wait()
            outs[d] = pltpu.async_copy(
                bufs[d % _NBUF], out_hbm.at[pl.ds(base + d * _CR, _CR)],
                osems[d % _NBUF])
    for d in range(_NCHUNK - _NBUF, _NCHUNK):
        outs[d].wait()


def kernel(input, weight):
    return _gather_kernel(input.astype(jnp.int32), weight)
